# Initial kernel scaffold; baseline (speedup 1.0000x reference)
import jax
import jax.numpy as jnp
from jax.experimental import pallas as pl


def kernel(x, edge_index, eps, W1, b1, g1, be1, W2, b2, g2, be2):
    raise NotImplementedError("write your pallas kernel here")


# speedup vs baseline: 2.6389x; 2.6389x over previous
import functools

import jax
import jax.numpy as jnp
from jax import lax
from jax.experimental import pallas as pl
from jax.experimental.pallas import tpu as pltpu
from jax.experimental.pallas import tpu_sc as plsc

N = 10000
E = 320000
D = 128
NL = 5

NC = 2
NS = 16
NW = NC * NS

CHUNK = 128
NCH = 80
SLOT = NCH * CHUNK
GRP = 16

NROWS = 10240
RPT = NROWS // NS

WIN = 240


def _worker_bounds():
    bounds = []
    half = E // 2
    nw = -(-half // WIN)
    per = [nw // NS + (1 if t < nw % NS else 0) for t in range(NS)]
    for c in range(NC):
        pos = c * half
        for t in range(NS):
            bounds.append(pos)
            pos = min(pos + per[t] * WIN, (c + 1) * half)
    bounds.append(E)
    return bounds


_BOUNDS = _worker_bounds()


def _sc_agg_body(h_hbm, src_hbm, dst_hbm, zeros_hbm, meta_hbm, out_hbm,
                 src_v, dst_v, rows0, rows1, midx_v, mrows_v, acc_sh, sem):
    c = lax.axis_index("c")
    s = lax.axis_index("s")
    w = c * NS + s

    pltpu.sync_copy(zeros_hbm, acc_sh.at[pl.ds(s * RPT, RPT)])
    plsc.subcore_barrier()

    def group(g, carry):
        base = w * NCH + g * GRP
        pltpu.sync_copy(src_hbm.at[pl.ds(base, GRP)], src_v)
        pltpu.sync_copy(dst_hbm.at[pl.ds(base, GRP)], dst_v)
        pltpu.async_copy(h_hbm.at[src_v.at[0]], rows0, sem)

        def pair(i, c2):
            j = 2 * i
            pltpu.async_copy(h_hbm.at[src_v.at[j + 1]], rows1, sem)
            pltpu.make_async_copy(h_hbm.at[src_v.at[j]], rows0, sem).wait()
            pltpu.sync_copy(rows0, acc_sh.at[dst_v.at[j]], add=True)

            @pl.when(j + 2 < GRP)
            def _():
                pltpu.async_copy(h_hbm.at[src_v.at[j + 2]], rows0, sem)

            pltpu.make_async_copy(h_hbm.at[src_v.at[j + 1]], rows1, sem).wait()
            pltpu.sync_copy(rows1, acc_sh.at[dst_v.at[j + 1]], add=True)
            return c2

        lax.fori_loop(0, GRP // 2, pair, carry)
        return carry

    lax.fori_loop(0, NCH // GRP, group, 0)

    plsc.subcore_barrier()

    @pl.when(s == 0)
    def _():
        pltpu.sync_copy(meta_hbm.at[c], midx_v)
        pltpu.sync_copy(acc_sh.at[pl.ds(N, 2 * NS)], mrows_v)
        pltpu.sync_copy(mrows_v, acc_sh.at[midx_v], add=True)

    plsc.subcore_barrier()
    pltpu.sync_copy(acc_sh.at[pl.ds(s * RPT, RPT)],
                    out_hbm.at[c, pl.ds(s * RPT, RPT)])


@functools.cache
def _get_sc_agg():
    return pl.kernel(
        _sc_agg_body,
        out_type=jax.ShapeDtypeStruct((NC, NROWS, D), jnp.float32),
        mesh=plsc.VectorSubcoreMesh(core_axis_name="c", subcore_axis_name="s",
                                    num_cores=NC, num_subcores=NS),
        scratch_types=[
            pltpu.VMEM((GRP, CHUNK), jnp.int32),
            pltpu.VMEM((GRP, CHUNK), jnp.int32),
            pltpu.VMEM((CHUNK, D), jnp.float32),
            pltpu.VMEM((CHUNK, D), jnp.float32),
            pltpu.VMEM((2 * NS,), jnp.int32),
            pltpu.VMEM((2 * NS, D), jnp.float32),
            pltpu.VMEM_SHARED((NROWS, D), jnp.float32),
            pltpu.SemaphoreType.DMA,
        ],
    )


def _sum_rows(ref, base, nblk, m=None):
    def body(i, acc):
        blk = ref[pl.ds(base + i * 8, 8), :]
        if m is not None:
            dd = blk - m
            blk = dd * dd
        return acc + blk

    acc = lax.fori_loop(0, nblk, body, jnp.zeros((8, D), jnp.float32))
    t4 = acc[0:4] + acc[4:8]
    t2 = t4[0:2] + t4[2:4]
    return t2[0:1] + t2[1:2]


def _mlp_body(h_ref, agg_ref, sc_ref, w1_ref, b1_ref, g1_ref, be1_ref,
              w2_ref, b2_ref, g2_ref, be2_ref, o_ref, u_scr, y_scr):
    agg = agg_ref[0, :N, :] + agg_ref[1, :N, :]
    z = h_ref[...] * sc_ref[...] + agg

    u_scr[...] = jnp.dot(z, w1_ref[...],
                         preferred_element_type=jnp.float32) + b1_ref[...]
    m = _sum_rows(u_scr, 0, N // 8) * 0.0001
    v = (_sum_rows(u_scr, 0, 625, m) + _sum_rows(u_scr, 5000, 625, m)) * 0.0001
    d = u_scr[...] - m
    a = g1_ref[...] * (d / jnp.sqrt(v + 1e-5)) + be1_ref[...]
    a = jnp.maximum(a, 0.0)

    y_scr[...] = jnp.dot(a, w2_ref[...],
                         preferred_element_type=jnp.float32) + b2_ref[...]
    m2 = _sum_rows(y_scr, 0, N // 8) * 0.0001
    v2 = (_sum_rows(y_scr, 0, 625, m2) + _sum_rows(y_scr, 5000, 625, m2)) * 0.0001
    d2 = y_scr[...] - m2
    o = g2_ref[...] * (d2 / jnp.sqrt(v2 + 1e-5)) + be2_ref[...]
    o_ref[...] = jnp.maximum(o, 0.0)


_mlp = pl.pallas_call(
    _mlp_body,
    out_shape=jax.ShapeDtypeStruct((N, D), jnp.float32),
    scratch_shapes=[pltpu.VMEM((N, D), jnp.float32),
                    pltpu.VMEM((N, D), jnp.float32)],
)


def _prepare_edges(src, dst):
    order = jnp.argsort(dst, stable=True)
    s_src = src[order]
    s_dst = dst[order]
    src_segs = []
    dst_segs = []
    fds = []
    lds = []
    for w in range(NW):
        lo, hi = _BOUNDS[w], _BOUNDS[w + 1]
        cnt = hi - lo
        sloc = w % NS
        seg_s = s_src[lo:hi]
        seg_d = s_dst[lo:hi]
        fd = seg_d[0]
        ld = seg_d[-1]
        fds.append(fd)
        lds.append(ld)
        seg_d = jnp.where(seg_d == fd, N + 2 * sloc,
                          jnp.where(seg_d == ld, N + 2 * sloc + 1, seg_d))
        src_segs.append(jnp.pad(seg_s, (0, SLOT - cnt)))
        dst_segs.append(jnp.pad(seg_d, (0, SLOT - cnt),
                                constant_values=N + 64 + w))
    srcp = jnp.concatenate(src_segs).reshape(NW * NCH, CHUNK)
    dstp = jnp.concatenate(dst_segs).reshape(NW * NCH, CHUNK)
    meta = jnp.stack([jnp.stack(fds), jnp.stack(lds)], axis=1)
    meta = meta.reshape(NC, NS * 2)
    return srcp, dstp, meta


def kernel(x, edge_index, eps, W1, b1, g1, be1, W2, b2, g2, be2):
    src = edge_index[0]
    dst = edge_index[1]
    srcp, dstp, meta = _prepare_edges(src, dst)
    zeros = jnp.zeros((RPT, D), jnp.float32)

    h = x
    outs = []
    for l in range(NL):
        agg2 = _get_sc_agg()(h, srcp, dstp, zeros, meta)
        sc = (1.0 + eps[l]).reshape(1, 1)
        h = _mlp(h, agg2, sc, W1[l], b1[l].reshape(1, D), g1[l].reshape(1, D),
                 be1[l].reshape(1, D), W2[l], b2[l].reshape(1, D),
                 g2[l].reshape(1, D), be2[l].reshape(1, D))
        outs.append(h)
    return tuple(outs)
